# Initial kernel scaffold; baseline (speedup 1.0000x reference)
import jax
import jax.numpy as jnp
from jax.experimental import pallas as pl


def kernel(sxl, idx_l, idx_r):
    raise NotImplementedError("write your pallas kernel here")


# speedup vs baseline: 6.3528x; 6.3528x over previous
import functools

import jax
import jax.numpy as jnp
from jax import lax
from jax.experimental import pallas as pl
from jax.experimental.pallas import tpu as pltpu
from jax.experimental.pallas import tpu_sc as plsc


def _gram_body(x_ref, g_ref):
    x = x_ref[0]
    g = lax.dot_general(x, x, (((1,), (1,)), ((), ())),
                        preferred_element_type=jnp.float32)
    g_ref[0] = g * (1.0 / x.shape[-1])


@functools.lru_cache(maxsize=None)
def _make_gram(BN, J, T):
    return pl.pallas_call(
        _gram_body,
        grid=(BN,),
        in_specs=[pl.BlockSpec((1, J, T), lambda i: (i, 0, 0))],
        out_specs=pl.BlockSpec((1, J, J), lambda i: (i, 0, 0)),
        out_shape=jax.ShapeDtypeStruct((BN, J, J), jnp.float32),
    )


@functools.lru_cache(maxsize=None)
def _make_pair_gather(BN, J, K):
    info = plsc.get_sparse_core_info()
    NC, NS = info.num_cores, info.num_subcores
    NW = NC * NS
    assert BN % NW == 0 and K % 16 == 0
    bn_per_w = BN // NW
    mesh = plsc.VectorSubcoreMesh(core_axis_name="c", subcore_axis_name="s")

    @functools.partial(
        pl.kernel,
        mesh=mesh,
        compiler_params=pltpu.CompilerParams(needs_layout_passes=False),
        out_type=jax.ShapeDtypeStruct((BN * K,), jnp.float32),
        scratch_types=[
            pltpu.VMEM((J * J,), jnp.float32),
            pltpu.VMEM((K,), jnp.int32),
            pltpu.VMEM((K,), jnp.int32),
            pltpu.VMEM((K,), jnp.float32),
        ],
    )
    def pair_gather(g_hbm, il_hbm, ir_hbm, out_hbm, g_v, il_v, ir_v, y_v):
        wid = lax.axis_index("s") * NC + lax.axis_index("c")
        pltpu.sync_copy(il_hbm, il_v)
        pltpu.sync_copy(ir_hbm, ir_v)
        for j in range(bn_per_w):
            bn = wid * bn_per_w + j
            pltpu.sync_copy(g_hbm.at[pl.ds(bn * (J * J), J * J)], g_v)
            for c in range(K // 16):
                il = il_v[pl.ds(c * 16, 16)]
                ir = ir_v[pl.ds(c * 16, 16)]
                y_v[pl.ds(c * 16, 16)] = plsc.load_gather(g_v, [il * J + ir])
            pltpu.sync_copy(y_v, out_hbm.at[pl.ds(bn * K, K)])

    return pair_gather


def kernel(sxl, idx_l, idx_r):
    B, N, J, A, T = sxl.shape
    K = idx_l.shape[0]
    BN = B * N
    x = sxl.reshape(BN, J * A, T)
    g = _make_gram(BN, J * A, T)(x).reshape(BN * J * A * J * A)
    y = _make_pair_gather(BN, J * A, K)(g, idx_l, idx_r)
    return y.reshape(B, N, K, 1)


# speedup vs baseline: 9.2895x; 1.4623x over previous
import functools

import jax
import jax.numpy as jnp
from jax import lax
from jax.experimental import pallas as pl
from jax.experimental.pallas import tpu as pltpu
from jax.experimental.pallas import tpu_sc as plsc


def _gram_body(x_ref, g_ref):
    x = x_ref[0, 0, :, 0, :]
    g = lax.dot_general(x, x, (((1,), (1,)), ((), ())),
                        preferred_element_type=jnp.float32)
    g_ref[0, 0] = g * (1.0 / x.shape[-1])


@functools.lru_cache(maxsize=None)
def _make_gram(B, N, J, A, T):
    return pl.pallas_call(
        _gram_body,
        grid=(B, N),
        in_specs=[pl.BlockSpec((1, 1, J, A, T), lambda b, n: (b, n, 0, 0, 0))],
        out_specs=pl.BlockSpec((1, 1, J, J), lambda b, n: (b, n, 0, 0)),
        out_shape=jax.ShapeDtypeStruct((B, N, J, J), jnp.float32),
    )


@functools.lru_cache(maxsize=None)
def _make_pair_gather(BN, J, K):
    info = plsc.get_sparse_core_info()
    NC, NS = info.num_cores, info.num_subcores
    NW = NC * NS
    assert BN % NW == 0 and K % 16 == 0
    bn_per_w = BN // NW
    mesh = plsc.VectorSubcoreMesh(core_axis_name="c", subcore_axis_name="s")

    @functools.partial(
        pl.kernel,
        mesh=mesh,
        compiler_params=pltpu.CompilerParams(needs_layout_passes=False),
        out_type=jax.ShapeDtypeStruct((BN * K,), jnp.float32),
        scratch_types=[
            pltpu.VMEM((J, J), jnp.float32),
            pltpu.VMEM((K,), jnp.int32),
            pltpu.VMEM((K,), jnp.int32),
            pltpu.VMEM((K,), jnp.float32),
        ],
    )
    def pair_gather(g_hbm, il_hbm, ir_hbm, out_hbm, g_v, il_v, ir_v, y_v):
        wid = lax.axis_index("s") * NC + lax.axis_index("c")
        pltpu.sync_copy(il_hbm, il_v)
        pltpu.sync_copy(ir_hbm, ir_v)
        for j in range(bn_per_w):
            bn = wid * bn_per_w + j
            pltpu.sync_copy(g_hbm.at[bn], g_v)
            for c in range(K // 16):
                il = il_v[pl.ds(c * 16, 16)]
                ir = ir_v[pl.ds(c * 16, 16)]
                y_v[pl.ds(c * 16, 16)] = plsc.load_gather(g_v, [il, ir])
            pltpu.sync_copy(y_v, out_hbm.at[pl.ds(bn * K, K)])

    return pair_gather


def kernel(sxl, idx_l, idx_r):
    B, N, J, A, T = sxl.shape
    K = idx_l.shape[0]
    BN = B * N
    g = _make_gram(B, N, J, A, T)(sxl)
    y = _make_pair_gather(BN, J, K)(g.reshape(BN, J, J), idx_l, idx_r)
    return y.reshape(B, N, K, 1)


# speedup vs baseline: 18.3609x; 1.9765x over previous
import functools

import jax
import jax.numpy as jnp
from jax import lax
from jax.experimental import pallas as pl
from jax.experimental.pallas import tpu as pltpu
from jax.experimental.pallas import tpu_sc as plsc

_STREAMS = 8
_HALVES = 2
_TL = 128


def _gram_body(*refs):
    x_refs, g_ref = refs[:-1], refs[-1]
    for s, x_ref in enumerate(x_refs):
        x3 = x_ref[0, 0]
        x = x3.reshape(x3.shape[0], x3.shape[1] * x3.shape[2])
        g = lax.dot_general(x, x, (((1,), (1,)), ((), ())),
                            preferred_element_type=jnp.float32)
        g_ref[s] = g * (1.0 / x.shape[-1])


@functools.lru_cache(maxsize=None)
def _make_gram(B, N, J, T, half, n_half):
    S = _STREAMS
    BN = B * N
    BNH = BN // n_half
    TC = T // _TL
    base = half * BNH

    def in_map(s):
        return lambda i: ((base + i * S + s) // N, (base + i * S + s) % N,
                          0, 0, 0)

    return pl.pallas_call(
        _gram_body,
        grid=(BNH // S,),
        in_specs=[pl.BlockSpec((1, 1, J, TC, _TL), in_map(s)) for s in range(S)],
        out_specs=pl.BlockSpec((S, J, J), lambda i: (i, 0, 0)),
        out_shape=jax.ShapeDtypeStruct((BNH, J, J), jnp.float32),
    )


@functools.lru_cache(maxsize=None)
def _make_pair_gather(BNH, J, K):
    info = plsc.get_sparse_core_info()
    NC, NS = info.num_cores, info.num_subcores
    NW = NC * NS
    assert BNH % NW == 0 and K % 16 == 0
    bn_per_w = BNH // NW
    mesh = plsc.VectorSubcoreMesh(core_axis_name="c", subcore_axis_name="s")

    @functools.partial(
        pl.kernel,
        mesh=mesh,
        compiler_params=pltpu.CompilerParams(needs_layout_passes=False),
        out_type=jax.ShapeDtypeStruct((BNH * K,), jnp.float32),
        scratch_types=[
            pltpu.VMEM((J, J), jnp.float32),
            pltpu.VMEM((K,), jnp.int32),
            pltpu.VMEM((K,), jnp.int32),
            pltpu.VMEM((K,), jnp.float32),
        ],
    )
    def pair_gather(g_hbm, il_hbm, ir_hbm, out_hbm, g_v, il_v, ir_v, y_v):
        wid = lax.axis_index("s") * NC + lax.axis_index("c")
        pltpu.sync_copy(il_hbm, il_v)
        pltpu.sync_copy(ir_hbm, ir_v)
        for j in range(bn_per_w):
            bn = wid * bn_per_w + j
            pltpu.sync_copy(g_hbm.at[bn], g_v)
            for c in range(K // 16):
                il = il_v[pl.ds(c * 16, 16)]
                ir = ir_v[pl.ds(c * 16, 16)]
                y_v[pl.ds(c * 16, 16)] = plsc.load_gather(g_v, [il, ir])
            pltpu.sync_copy(y_v, out_hbm.at[pl.ds(bn * K, K)])

    return pair_gather


def kernel(sxl, idx_l, idx_r):
    B, N, J, A, T = sxl.shape
    K = idx_l.shape[0]
    BN = B * N
    BNH = BN // _HALVES
    x5 = sxl.reshape(B, N, J * A, T // _TL, _TL)
    ys = []
    for h in range(_HALVES):
        g = _make_gram(B, N, J * A, T, h, _HALVES)(*([x5] * _STREAMS))
        ys.append(_make_pair_gather(BNH, J * A, K)(g, idx_l, idx_r))
    y = jnp.concatenate(ys)
    return y.reshape(B, N, K, 1)


# speedup vs baseline: 19.3809x; 1.0556x over previous
import functools

import jax
import jax.numpy as jnp
from jax import lax
from jax.experimental import pallas as pl
from jax.experimental.pallas import tpu as pltpu
from jax.experimental.pallas import tpu_sc as plsc

_STREAMS = 8
_TL = 128


def _gram_body(*refs):
    x_refs, g_ref = refs[:-1], refs[-1]
    for s, x_ref in enumerate(x_refs):
        x3 = x_ref[0, 0]
        x = x3.reshape(x3.shape[0], x3.shape[1] * x3.shape[2])
        g = lax.dot_general(x, x, (((1,), (1,)), ((), ())),
                            preferred_element_type=jnp.float32)
        g_ref[s] = g * (1.0 / x.shape[-1])


@functools.lru_cache(maxsize=None)
def _make_gram(B, N, J, T):
    S = _STREAMS
    BN = B * N
    TC = T // _TL

    def in_map(s):
        return lambda i: ((i * S + s) // N, (i * S + s) % N, 0, 0, 0)

    return pl.pallas_call(
        _gram_body,
        grid=(BN // S,),
        in_specs=[pl.BlockSpec((1, 1, J, TC, _TL), in_map(s)) for s in range(S)],
        out_specs=pl.BlockSpec((S, J, J), lambda i: (i, 0, 0)),
        out_shape=pltpu.MemorySpace.HBM((BN, J, J), jnp.float32),
    )


@functools.lru_cache(maxsize=None)
def _make_pair_gather(BN, J, K):
    info = plsc.get_sparse_core_info()
    NC, NS = info.num_cores, info.num_subcores
    NW = NC * NS
    assert BN % NW == 0 and K % 16 == 0
    bn_per_w = BN // NW
    mesh = plsc.VectorSubcoreMesh(core_axis_name="c", subcore_axis_name="s")

    @functools.partial(
        pl.kernel,
        mesh=mesh,
        compiler_params=pltpu.CompilerParams(needs_layout_passes=False),
        out_type=jax.ShapeDtypeStruct((BN * K,), jnp.float32),
        scratch_types=[
            pltpu.VMEM((J, J), jnp.float32),
            pltpu.VMEM((K,), jnp.int32),
            pltpu.VMEM((K,), jnp.int32),
            pltpu.VMEM((K,), jnp.float32),
        ],
    )
    def pair_gather(g_hbm, il_hbm, ir_hbm, out_hbm, g_v, il_v, ir_v, y_v):
        wid = lax.axis_index("s") * NC + lax.axis_index("c")
        pltpu.sync_copy(il_hbm, il_v)
        pltpu.sync_copy(ir_hbm, ir_v)
        for j in range(bn_per_w):
            bn = wid * bn_per_w + j
            pltpu.sync_copy(g_hbm.at[bn], g_v)
            for c in range(K // 16):
                il = il_v[pl.ds(c * 16, 16)]
                ir = ir_v[pl.ds(c * 16, 16)]
                y_v[pl.ds(c * 16, 16)] = plsc.load_gather(g_v, [il, ir])
            pltpu.sync_copy(y_v, out_hbm.at[pl.ds(bn * K, K)])

    return pair_gather


def kernel(sxl, idx_l, idx_r):
    B, N, J, A, T = sxl.shape
    K = idx_l.shape[0]
    BN = B * N
    x5 = sxl.reshape(B, N, J * A, T // _TL, _TL)
    g = _make_gram(B, N, J * A, T)(*([x5] * _STREAMS))
    y = _make_pair_gather(BN, J * A, K)(g, idx_l, idx_r)
    return y.reshape(B, N, K, 1)


# speedup vs baseline: 20.2329x; 1.0440x over previous
import functools

import jax
import jax.numpy as jnp
from jax import lax
from jax.experimental import pallas as pl
from jax.experimental.pallas import tpu as pltpu
from jax.experimental.pallas import tpu_sc as plsc

_STREAMS = 16
_TL = 128


def _gram_body(*refs):
    x_refs, g_ref = refs[:-1], refs[-1]
    for s, x_ref in enumerate(x_refs):
        x3 = x_ref[0, 0]
        x = x3.reshape(x3.shape[0], x3.shape[1] * x3.shape[2])
        g = lax.dot_general(x, x, (((1,), (1,)), ((), ())),
                            preferred_element_type=jnp.float32)
        g_ref[s] = g * (1.0 / x.shape[-1])


@functools.lru_cache(maxsize=None)
def _make_gram(B, N, J, T):
    S = _STREAMS
    BN = B * N
    TC = T // _TL

    def in_map(s):
        return lambda i: ((i * S + s) // N, (i * S + s) % N, 0, 0, 0)

    return pl.pallas_call(
        _gram_body,
        grid=(BN // S,),
        in_specs=[pl.BlockSpec((1, 1, J, TC, _TL), in_map(s)) for s in range(S)],
        out_specs=pl.BlockSpec((S, J, J), lambda i: (i, 0, 0)),
        out_shape=pltpu.MemorySpace.HBM((BN, J, J), jnp.float32),
    )


@functools.lru_cache(maxsize=None)
def _make_pair_gather(BN, J, K):
    info = plsc.get_sparse_core_info()
    NC, NS = info.num_cores, info.num_subcores
    NW = NC * NS
    assert BN % NW == 0 and K % 16 == 0
    bn_per_w = BN // NW
    mesh = plsc.VectorSubcoreMesh(core_axis_name="c", subcore_axis_name="s")

    @functools.partial(
        pl.kernel,
        mesh=mesh,
        compiler_params=pltpu.CompilerParams(needs_layout_passes=False),
        out_type=jax.ShapeDtypeStruct((BN * K,), jnp.float32),
        scratch_types=[
            pltpu.VMEM((J, J), jnp.float32),
            pltpu.VMEM((K,), jnp.int32),
            pltpu.VMEM((K,), jnp.int32),
            pltpu.VMEM((K,), jnp.float32),
        ],
    )
    def pair_gather(g_hbm, il_hbm, ir_hbm, out_hbm, g_v, il_v, ir_v, y_v):
        wid = lax.axis_index("s") * NC + lax.axis_index("c")
        pltpu.sync_copy(il_hbm, il_v)
        pltpu.sync_copy(ir_hbm, ir_v)
        for j in range(bn_per_w):
            bn = wid * bn_per_w + j
            pltpu.sync_copy(g_hbm.at[bn], g_v)
            for c in range(K // 16):
                il = il_v[pl.ds(c * 16, 16)]
                ir = ir_v[pl.ds(c * 16, 16)]
                y_v[pl.ds(c * 16, 16)] = plsc.load_gather(g_v, [il, ir])
            pltpu.sync_copy(y_v, out_hbm.at[pl.ds(bn * K, K)])

    return pair_gather


def kernel(sxl, idx_l, idx_r):
    B, N, J, A, T = sxl.shape
    K = idx_l.shape[0]
    BN = B * N
    x5 = sxl.reshape(B, N, J * A, T // _TL, _TL)
    g = _make_gram(B, N, J * A, T)(*([x5] * _STREAMS))
    y = _make_pair_gather(BN, J * A, K)(g, idx_l, idx_r)
    return y.reshape(B, N, K, 1)
